# baseline (speedup 1.0000x reference)
import functools

import jax
import jax.numpy as jnp
from jax import lax
from jax.experimental import pallas as pl
from jax.experimental.pallas import tpu as pltpu
from jax.experimental.pallas import tpu_sc as plsc

EMBED = 64
CHUNK = 512


@functools.partial(jax.jit, static_argnames=("n_total",))
def _emb_lookup(x_flat3, w, n_total):
    info = plsc.get_sparse_core_info()
    nw = info.num_cores * info.num_subcores
    per_w = n_total // nw
    n_chunks = per_w // CHUNK

    mesh = plsc.VectorSubcoreMesh(core_axis_name="c", subcore_axis_name="s")

    @functools.partial(
        pl.kernel,
        mesh=mesh,
        out_type=jax.ShapeDtypeStruct((n_total, EMBED), jnp.float32),
        scratch_types=[
            pltpu.VMEM((n_chunks, CHUNK), jnp.int32),
            pltpu.VMEM((CHUNK, EMBED), jnp.float32),
            pltpu.SemaphoreType.DMA,
        ],
        compiler_params=pltpu.CompilerParams(use_tc_tiling_on_sc=False),
    )
    def k(x_hbm, w_hbm, out_hbm, idx_v, rows_v, sem):
        wid = lax.axis_index("s") * info.num_cores + lax.axis_index("c")
        base = wid * per_w
        pltpu.sync_copy(x_hbm.at[wid], idx_v)

        def body(j, _):
            pltpu.async_copy(w_hbm.at[idx_v.at[j]], rows_v, sem).wait()
            pltpu.sync_copy(rows_v, out_hbm.at[pl.ds(base + j * CHUNK, CHUNK)])
            return 0

        lax.fori_loop(0, n_chunks, body, 0)

    return k(x_flat3, w)


def kernel(x, W_embed):
    n_total = x.shape[0] * x.shape[1]
    info = plsc.get_sparse_core_info()
    nw = info.num_cores * info.num_subcores
    x_flat3 = x.reshape(nw, n_total // nw // CHUNK, CHUNK).astype(jnp.int32)
    out = _emb_lookup(x_flat3, W_embed, n_total)
    return out.reshape(x.shape[0], x.shape[1], EMBED)


# speedup vs baseline: 1.0228x; 1.0228x over previous
import functools

import jax
import jax.numpy as jnp
from jax import lax
from jax.experimental import pallas as pl
from jax.experimental.pallas import tpu as pltpu
from jax.experimental.pallas import tpu_sc as plsc

EMBED = 64
CHUNK = 512


@functools.partial(jax.jit, static_argnames=("n_total",))
def _emb_lookup(x_flat3, w, n_total):
    info = plsc.get_sparse_core_info()
    nw = info.num_cores * info.num_subcores
    per_w = n_total // nw
    n_chunks = per_w // CHUNK
    assert n_chunks % 2 == 0

    mesh = plsc.VectorSubcoreMesh(core_axis_name="c", subcore_axis_name="s")

    @functools.partial(
        pl.kernel,
        mesh=mesh,
        out_type=jax.ShapeDtypeStruct((n_total, EMBED), jnp.float32),
        scratch_types=[
            pltpu.VMEM((n_chunks, CHUNK), jnp.int32),
            pltpu.VMEM((CHUNK, EMBED), jnp.float32),
            pltpu.VMEM((CHUNK, EMBED), jnp.float32),
            pltpu.SemaphoreType.DMA,
            pltpu.SemaphoreType.DMA,
        ],
        compiler_params=pltpu.CompilerParams(use_tc_tiling_on_sc=False),
    )
    def k(x_hbm, w_hbm, out_hbm, idx_v, rows0, rows1, sem0, sem1):
        wid = lax.axis_index("s") * info.num_cores + lax.axis_index("c")
        base = wid * per_w
        pltpu.sync_copy(x_hbm.at[wid], idx_v)

        rows = (rows0, rows1)
        sems = (sem0, sem1)
        pltpu.async_copy(w_hbm.at[idx_v.at[0]], rows0, sem0)

        def pair_body(g, _):
            for b in range(2):
                j = 2 * g + b
                pltpu.make_async_copy(
                    w_hbm.at[idx_v.at[j]], rows[b], sems[b]
                ).wait()

                @pl.when(j + 1 < n_chunks)
                def _():
                    pltpu.async_copy(
                        w_hbm.at[idx_v.at[j + 1]], rows[1 - b], sems[1 - b]
                    )

                pltpu.sync_copy(
                    rows[b], out_hbm.at[pl.ds(base + j * CHUNK, CHUNK)]
                )
            return 0

        lax.fori_loop(0, n_chunks // 2, pair_body, 0)

    return k(x_flat3, w)


def kernel(x, W_embed):
    n_total = x.shape[0] * x.shape[1]
    info = plsc.get_sparse_core_info()
    nw = info.num_cores * info.num_subcores
    x_flat3 = x.reshape(nw, n_total // nw // CHUNK, CHUNK).astype(jnp.int32)
    out = _emb_lookup(x_flat3, W_embed, n_total)
    return out.reshape(x.shape[0], x.shape[1], EMBED)
